# baseline (speedup 1.0000x reference)
import jax
import jax.numpy as jnp
from jax import lax
from jax.experimental import pallas as pl
from jax.experimental.pallas import tpu as pltpu
from jax.experimental.pallas import tpu_sc as plsc

B = 16384
EMB = 32
NROW = 1000000
NW = 32
BPW = B // NW
CHUNK = 128
NCHUNK = BPW // CHUNK


def _sc_kernel(user_ids_h, pos_ids_h, neg_ids_h, user_flat_h, movie_flat_h,
               out_pos_h, out_neg_h,
               uidx_v, pidx_v, nidx_v,
               u_b0, u_b1, p_b0, p_b1, n_b0, n_b1,
               op_v, on_v, sem0, sem1):
    wid = lax.axis_index("s") * 2 + lax.axis_index("c")

    pltpu.sync_copy(user_ids_h.at[pl.ds(wid * BPW, BPW)], uidx_v.at[...])
    pltpu.sync_copy(pos_ids_h.at[pl.ds(wid * BPW, BPW)], pidx_v.at[...])
    pltpu.sync_copy(neg_ids_h.at[pl.ds(wid * BPW, BPW)], nidx_v.at[...])

    bufs = ((u_b0, p_b0, n_b0), (u_b1, p_b1, n_b1))
    sems = (sem0, sem1)
    tabs = (user_flat_h, movie_flat_h, movie_flat_h)
    idxs = (uidx_v, pidx_v, nidx_v)

    def fire_chunk(j):
        sem = sems[j % 2]
        for t in range(3):
            idx = idxs[t].at[pl.ds(j * CHUNK, CHUNK)]
            buf = bufs[j % 2][t]
            for d in range(EMB):
                src = tabs[t].at[pl.ds(d * NROW, NROW)].at[idx]
                pltpu.async_copy(src, buf.at[d], sem)

    def wait_chunk(j):
        sem = sems[j % 2]
        for t in range(3):
            pltpu.make_async_copy(
                user_flat_h.at[pl.ds(0, EMB * CHUNK)],
                bufs[j % 2][t].at[...],
                sem,
            ).wait()

    def compute_chunk(j):
        u_b, p_b, n_b = bufs[j % 2]

        def block(blk, carry):
            cols = pl.ds(blk * 16, 16)
            accp = jnp.zeros((16,), jnp.float32)
            accn = jnp.zeros((16,), jnp.float32)
            for d in range(EMB):
                ug = u_b[d, cols]
                accp = accp + ug * p_b[d, cols]
                accn = accn + ug * n_b[d, cols]
            base = j * CHUNK + blk * 16
            op_v[pl.ds(base, 16)] = accp
            on_v[pl.ds(base, 16)] = accn
            return carry

        lax.fori_loop(0, CHUNK // 16, block, 0)

    fire_chunk(0)
    for j in range(NCHUNK):
        if j + 1 < NCHUNK:
            fire_chunk(j + 1)
        wait_chunk(j)
        compute_chunk(j)

    out = pl.ds(wid * BPW, BPW)
    pltpu.sync_copy(op_v, out_pos_h.at[out])
    pltpu.sync_copy(on_v, out_neg_h.at[out])


@jax.jit
def kernel(user_ids, pos_ids, neg_ids, user_emb, movie_emb):
    uids = user_ids.astype(jnp.int32)
    pids = pos_ids.astype(jnp.int32)
    nids = neg_ids.astype(jnp.int32)
    user_flat = user_emb.T.reshape(-1)
    movie_flat = movie_emb.T.reshape(-1)

    mesh = plsc.VectorSubcoreMesh(
        core_axis_name="c", subcore_axis_name="s", num_cores=2, num_subcores=16
    )
    buf = pltpu.VMEM((EMB, CHUNK), jnp.float32)
    run = pl.kernel(
        _sc_kernel,
        out_type=(
            jax.ShapeDtypeStruct((B,), jnp.float32),
            jax.ShapeDtypeStruct((B,), jnp.float32),
        ),
        mesh=mesh,
        scratch_types=[
            pltpu.VMEM((BPW,), jnp.int32),
            pltpu.VMEM((BPW,), jnp.int32),
            pltpu.VMEM((BPW,), jnp.int32),
            buf, buf, buf, buf, buf, buf,
            pltpu.VMEM((BPW,), jnp.float32),
            pltpu.VMEM((BPW,), jnp.float32),
            pltpu.SemaphoreType.DMA,
            pltpu.SemaphoreType.DMA,
        ],
        compiler_params=pltpu.CompilerParams(
            needs_layout_passes=False, use_tc_tiling_on_sc=False
        ),
    )
    return run(uids, pids, nids, user_flat, movie_flat)


# speedup vs baseline: 7.7772x; 7.7772x over previous
import jax
import jax.numpy as jnp
from jax import lax
from jax.experimental import pallas as pl
from jax.experimental.pallas import tpu as pltpu
from jax.experimental.pallas import tpu_sc as plsc

B = 16384
EMB = 32
NROW = 1000000
NW = 32
BPW = B // NW
CHUNK = 128
NCHUNK = BPW // CHUNK


def _sc_kernel(user_ids_h, pos_ids_h, neg_ids_h, user_emb_h, movie_emb_h,
               out_pos_h, out_neg_h,
               uidx_v, pidx_v, nidx_v,
               u_b0, u_b1, p_b0, p_b1, n_b0, n_b1,
               op_v, on_v, *sems_flat):
    wid = lax.axis_index("s") * 2 + lax.axis_index("c")

    pltpu.sync_copy(user_ids_h.at[pl.ds(wid * BPW, BPW)], uidx_v)
    pltpu.sync_copy(pos_ids_h.at[pl.ds(wid * BPW, BPW)], pidx_v)
    pltpu.sync_copy(neg_ids_h.at[pl.ds(wid * BPW, BPW)], nidx_v)

    bufs = ((u_b0, p_b0, n_b0), (u_b1, p_b1, n_b1))
    NSEM = 8
    sems = (sems_flat[:NSEM], sems_flat[NSEM:])

    def fire_chunk(j):
        u_b, p_b, n_b = bufs[j % 2]
        psems = sems[j % 2]

        def grp(g, carry):
            src = pl.ds(j * CHUNK + g * 16, 16)
            uids16 = uidx_v[src]
            pids16 = pidx_v[src]
            nids16 = nidx_v[src]
            for k in range(16):
                dst = g * 16 + k
                pltpu.async_copy(
                    user_emb_h.at[pl.ds(uids16[k], 1)], u_b.at[dst],
                    psems[(3 * k) % NSEM])
                pltpu.async_copy(
                    movie_emb_h.at[pl.ds(pids16[k], 1)], p_b.at[dst],
                    psems[(3 * k + 1) % NSEM])
                pltpu.async_copy(
                    movie_emb_h.at[pl.ds(nids16[k], 1)], n_b.at[dst],
                    psems[(3 * k + 2) % NSEM])
            return carry

        lax.fori_loop(0, CHUNK // 16, grp, 0)

    def wait_chunk(j):
        psems = sems[j % 2]
        per_sem = 3 * CHUNK // NSEM
        for s in range(NSEM):
            pltpu.make_async_copy(
                user_emb_h.at[pl.ds(0, per_sem)],
                bufs[j % 2][0].at[pl.ds(0, per_sem), 0],
                psems[s],
            ).wait()

    def compute_chunk(j):
        u_b, p_b, n_b = bufs[j % 2]

        def block(blk, carry):
            rows = blk * 16 + lax.iota(jnp.int32, 16)
            zero = jnp.zeros((16,), jnp.int32)
            accp = jnp.zeros((16,), jnp.float32)
            accn = jnp.zeros((16,), jnp.float32)
            for d in range(EMB):
                col = jnp.full((16,), d, jnp.int32)
                ug = plsc.load_gather(u_b, [rows, zero, col])
                pg = plsc.load_gather(p_b, [rows, zero, col])
                ng = plsc.load_gather(n_b, [rows, zero, col])
                accp = accp + ug * pg
                accn = accn + ug * ng
            base = j * CHUNK + blk * 16
            op_v[pl.ds(base, 16)] = accp
            on_v[pl.ds(base, 16)] = accn
            return carry

        lax.fori_loop(0, CHUNK // 16, block, 0)

    fire_chunk(0)
    for j in range(NCHUNK):
        if j + 1 < NCHUNK:
            fire_chunk(j + 1)
        wait_chunk(j)
        compute_chunk(j)

    out = pl.ds(wid * BPW, BPW)
    pltpu.sync_copy(op_v, out_pos_h.at[out])
    pltpu.sync_copy(on_v, out_neg_h.at[out])


_TBLK = 4096


def _tc_transpose(tab_t):
    grid = (NROW + _TBLK - 1) // _TBLK

    def body(in_ref, out_ref):
        out_ref[...] = in_ref[...].T

    return pl.pallas_call(
        body,
        grid=(grid,),
        in_specs=[pl.BlockSpec((EMB, _TBLK), lambda j: (0, j))],
        out_specs=pl.BlockSpec((_TBLK, EMB), lambda j: (j, 0)),
        out_shape=jax.ShapeDtypeStruct((NROW, EMB), jnp.float32),
    )(tab_t)


@jax.jit
def kernel(user_ids, pos_ids, neg_ids, user_emb, movie_emb):
    uids = user_ids.astype(jnp.int32)
    pids = pos_ids.astype(jnp.int32)
    nids = neg_ids.astype(jnp.int32)
    user_rm = _tc_transpose(user_emb.T)
    movie_rm = _tc_transpose(movie_emb.T)

    mesh = plsc.VectorSubcoreMesh(
        core_axis_name="c", subcore_axis_name="s", num_cores=2, num_subcores=16
    )
    buf = pltpu.VMEM((CHUNK, 1, EMB), jnp.float32)
    run = pl.kernel(
        _sc_kernel,
        out_type=(
            jax.ShapeDtypeStruct((B,), jnp.float32),
            jax.ShapeDtypeStruct((B,), jnp.float32),
        ),
        mesh=mesh,
        scratch_types=[
            pltpu.VMEM((BPW,), jnp.int32),
            pltpu.VMEM((BPW,), jnp.int32),
            pltpu.VMEM((BPW,), jnp.int32),
            buf, buf, buf, buf, buf, buf,
            pltpu.VMEM((BPW,), jnp.float32),
            pltpu.VMEM((BPW,), jnp.float32),
        ] + [pltpu.SemaphoreType.DMA] * 16,
        compiler_params=pltpu.CompilerParams(needs_layout_passes=False),
    )
    return run(uids, pids, nids, user_rm, movie_rm)


# speedup vs baseline: 8.2812x; 1.0648x over previous
import jax
import jax.numpy as jnp
from jax import lax
from jax.experimental import pallas as pl
from jax.experimental.pallas import tpu as pltpu
from jax.experimental.pallas import tpu_sc as plsc

B = 16384
EMB = 32
NROW = 1000000
NW = 32
BPW = B // NW
CHUNK = 128
NCHUNK = BPW // CHUNK


def _sc_kernel(user_ids_h, pos_ids_h, neg_ids_h, user_emb_h, movie_emb_h,
               out_pos_h, out_neg_h,
               uidx_v, pidx_v, nidx_v,
               u_b0, u_b1, p_b0, p_b1, n_b0, n_b1,
               op_v, on_v, *sems_flat):
    wid = lax.axis_index("s") * 2 + lax.axis_index("c")

    pltpu.sync_copy(user_ids_h.at[pl.ds(wid * BPW, BPW)], uidx_v)
    pltpu.sync_copy(pos_ids_h.at[pl.ds(wid * BPW, BPW)], pidx_v)
    pltpu.sync_copy(neg_ids_h.at[pl.ds(wid * BPW, BPW)], nidx_v)

    bufs = ((u_b0, p_b0, n_b0), (u_b1, p_b1, n_b1))
    NSEM = 8
    sems = (sems_flat[:NSEM], sems_flat[NSEM:])

    def fire_chunk(j):
        u_b, p_b, n_b = bufs[j % 2]
        psems = sems[j % 2]

        def grp(g, carry):
            src = pl.ds(j * CHUNK + g * 16, 16)
            uids16 = uidx_v[src]
            pids16 = pidx_v[src]
            nids16 = nidx_v[src]
            for k in range(16):
                dst = g * 16 + k
                pltpu.async_copy(
                    user_emb_h.at[pl.ds(uids16[k], 1)], u_b.at[dst],
                    psems[(3 * k) % NSEM])
                pltpu.async_copy(
                    movie_emb_h.at[pl.ds(pids16[k], 1)], p_b.at[dst],
                    psems[(3 * k + 1) % NSEM])
                pltpu.async_copy(
                    movie_emb_h.at[pl.ds(nids16[k], 1)], n_b.at[dst],
                    psems[(3 * k + 2) % NSEM])
            return carry

        lax.fori_loop(0, CHUNK // 16, grp, 0)

    def wait_chunk(j):
        psems = sems[j % 2]
        per_sem = 3 * CHUNK // NSEM
        for s in range(NSEM):
            pltpu.make_async_copy(
                user_emb_h.at[pl.ds(0, per_sem)],
                bufs[j % 2][0].at[pl.ds(0, per_sem), 0],
                psems[s],
            ).wait()

    def compute_chunk(j):
        u_b, p_b, n_b = bufs[j % 2]

        def block(blk, carry):
            rows = blk * 16 + lax.iota(jnp.int32, 16)
            zero = jnp.zeros((16,), jnp.int32)
            accp = jnp.zeros((16,), jnp.float32)
            accn = jnp.zeros((16,), jnp.float32)
            for d in range(EMB):
                col = jnp.full((16,), d, jnp.int32)
                ug = plsc.load_gather(u_b, [rows, zero, col])
                pg = plsc.load_gather(p_b, [rows, zero, col])
                ng = plsc.load_gather(n_b, [rows, zero, col])
                accp = accp + ug * pg
                accn = accn + ug * ng
            base = j * CHUNK + blk * 16
            op_v[pl.ds(base, 16)] = accp
            on_v[pl.ds(base, 16)] = accn
            return carry

        lax.fori_loop(0, CHUNK // 16, block, 0)

    fire_chunk(0)
    for j in range(NCHUNK):
        if j + 1 < NCHUNK:
            fire_chunk(j + 1)
        wait_chunk(j)
        compute_chunk(j)

    out = pl.ds(wid * BPW, BPW)
    pltpu.sync_copy(op_v, out_pos_h.at[out])
    pltpu.sync_copy(on_v, out_neg_h.at[out])


_TBLK = 4096


def _tc_transpose(tab_t):
    grid = (NROW + _TBLK - 1) // _TBLK

    def body(in_ref, out_ref):
        out_ref[...] = in_ref[...].T

    return pl.pallas_call(
        body,
        grid=(grid,),
        in_specs=[pl.BlockSpec((EMB, _TBLK), lambda j: (0, j))],
        out_specs=pl.BlockSpec((_TBLK, EMB), lambda j: (j, 0)),
        out_shape=jax.ShapeDtypeStruct((NROW, EMB), jnp.float32),
    )(tab_t)


@jax.jit
def kernel(user_ids, pos_ids, neg_ids, user_emb, movie_emb):
    uids = user_ids.astype(jnp.int32)
    pids = pos_ids.astype(jnp.int32)
    nids = neg_ids.astype(jnp.int32)
    user_rm = user_emb
    movie_rm = movie_emb

    mesh = plsc.VectorSubcoreMesh(
        core_axis_name="c", subcore_axis_name="s", num_cores=2, num_subcores=16
    )
    buf = pltpu.VMEM((CHUNK, 1, EMB), jnp.float32)
    run = pl.kernel(
        _sc_kernel,
        out_type=(
            jax.ShapeDtypeStruct((B,), jnp.float32),
            jax.ShapeDtypeStruct((B,), jnp.float32),
        ),
        mesh=mesh,
        scratch_types=[
            pltpu.VMEM((BPW,), jnp.int32),
            pltpu.VMEM((BPW,), jnp.int32),
            pltpu.VMEM((BPW,), jnp.int32),
            buf, buf, buf, buf, buf, buf,
            pltpu.VMEM((BPW,), jnp.float32),
            pltpu.VMEM((BPW,), jnp.float32),
        ] + [pltpu.SemaphoreType.DMA] * 16,
        compiler_params=pltpu.CompilerParams(needs_layout_passes=False),
    )
    return run(uids, pids, nids, user_rm, movie_rm)


# speedup vs baseline: 15.2769x; 1.8448x over previous
import jax
import jax.numpy as jnp
from jax import lax
from jax.experimental import pallas as pl
from jax.experimental.pallas import tpu as pltpu
from jax.experimental.pallas import tpu_sc as plsc

B = 16384
EMB = 32
NROW = 1000000
NW = 32
BPW = B // NW
GRP = 4
NGRP = BPW // GRP
NSEM = 6


def _sc_kernel(user_ids_h, pos_ids_h, neg_ids_h, user_t_h, movie_t_h,
               out_pos_h, out_neg_h,
               uidx_v, pidx_v, nidx_v,
               w_b0, w_b1, op_v, on_v, *sems_flat):
    wid = lax.axis_index("s") * 2 + lax.axis_index("c")

    pltpu.sync_copy(user_ids_h.at[pl.ds(wid * BPW, BPW)],
                    uidx_v.at[pl.ds(0, BPW)])
    pltpu.sync_copy(pos_ids_h.at[pl.ds(wid * BPW, BPW)],
                    pidx_v.at[pl.ds(0, BPW)])
    pltpu.sync_copy(neg_ids_h.at[pl.ds(wid * BPW, BPW)],
                    nidx_v.at[pl.ds(0, BPW)])

    bufs = (w_b0, w_b1)
    sems = (sems_flat[:NSEM], sems_flat[NSEM:])
    tabs = (user_t_h, movie_t_h, movie_t_h)
    idxs = (uidx_v, pidx_v, nidx_v)

    def fire_group(g, par):
        buf = bufs[par]
        psems = sems[par]
        vecs = [idxs[t][pl.ds(g * GRP, 16)] for t in range(3)]
        for t in range(3):
            for k in range(GRP):
                rid = vecs[t][k]
                col0 = pl.multiple_of((rid // 128) * 128, 128)
                j = t * GRP + k
                pltpu.async_copy(
                    tabs[t].at[pl.ds(0, EMB), pl.ds(col0, 128)],
                    buf.at[j],
                    psems[j % NSEM])

    def wait_group(par):
        buf = bufs[par]
        psems = sems[par]
        for j in range(3 * GRP):
            pltpu.make_async_copy(
                tabs[0].at[pl.ds(0, EMB), pl.ds(0, 128)],
                buf.at[j],
                psems[j % NSEM],
            ).wait()

    lo = lax.iota(jnp.int32, 16)
    hi = lo + 16

    def extract(buf, slot, cid):
        col = jnp.full((16,), cid, jnp.int32)
        slot_v = jnp.full((16,), slot, jnp.int32)
        a = plsc.load_gather(buf, [slot_v, lo, col])
        b = plsc.load_gather(buf, [slot_v, hi, col])
        return a, b

    def compute_group(g, par, accp, accn):
        buf = bufs[par]
        vecs = [idxs[t][pl.ds(g * GRP, 16)] for t in range(3)]
        for k in range(GRP):
            ua, ub = extract(buf, 0 * GRP + k, lax.rem(vecs[0][k], 128))
            pa, pb = extract(buf, 1 * GRP + k, lax.rem(vecs[1][k], 128))
            na, nb = extract(buf, 2 * GRP + k, lax.rem(vecs[2][k], 128))
            lane = lax.rem(g, 4) * GRP + k
            accp = jnp.where(lo == lane, jnp.sum(ua * pa + ub * pb), accp)
            accn = jnp.where(lo == lane, jnp.sum(ua * na + ub * nb), accn)
        is_last = lax.rem(g, 4) == 3

        @pl.when(is_last)
        def _():
            base = (g // 4) * 16
            op_v[pl.ds(base, 16)] = accp
            on_v[pl.ds(base, 16)] = accn

        keep = jnp.where(is_last, 0.0, 1.0)
        return accp * keep, accn * keep

    fire_group(0, 0)
    zeros = jnp.zeros((16,), jnp.float32)

    def pair(i, carry):
        accp, accn = carry
        g0 = i * 2
        fire_group(g0 + 1, 1)
        wait_group(0)
        accp, accn = compute_group(g0, 0, accp, accn)

        @pl.when(i + 1 < NGRP // 2)
        def _():
            fire_group(g0 + 2, 0)

        wait_group(1)
        accp, accn = compute_group(g0 + 1, 1, accp, accn)
        return accp, accn

    lax.fori_loop(0, NGRP // 2, pair, (zeros, zeros))

    out = pl.ds(wid * BPW, BPW)
    pltpu.sync_copy(op_v, out_pos_h.at[out])
    pltpu.sync_copy(on_v, out_neg_h.at[out])


@jax.jit
def kernel(user_ids, pos_ids, neg_ids, user_emb, movie_emb):
    uids = user_ids.astype(jnp.int32)
    pids = pos_ids.astype(jnp.int32)
    nids = neg_ids.astype(jnp.int32)

    mesh = plsc.VectorSubcoreMesh(
        core_axis_name="c", subcore_axis_name="s", num_cores=2, num_subcores=16
    )
    buf = pltpu.VMEM((3 * GRP, EMB, 128), jnp.float32)
    run = pl.kernel(
        _sc_kernel,
        out_type=(
            jax.ShapeDtypeStruct((B,), jnp.float32),
            jax.ShapeDtypeStruct((B,), jnp.float32),
        ),
        mesh=mesh,
        scratch_types=[
            pltpu.VMEM((BPW + 16,), jnp.int32),
            pltpu.VMEM((BPW + 16,), jnp.int32),
            pltpu.VMEM((BPW + 16,), jnp.int32),
            buf, buf,
            pltpu.VMEM((BPW,), jnp.float32),
            pltpu.VMEM((BPW,), jnp.float32),
        ] + [pltpu.SemaphoreType.DMA] * (2 * NSEM),
        compiler_params=pltpu.CompilerParams(needs_layout_passes=False),
    )
    return run(uids, pids, nids, user_emb.T, movie_emb.T)


# speedup vs baseline: 15.5840x; 1.0201x over previous
import jax
import jax.numpy as jnp
from jax import lax
from jax.experimental import pallas as pl
from jax.experimental.pallas import tpu as pltpu
from jax.experimental.pallas import tpu_sc as plsc

B = 16384
EMB = 32
NROW = 1000000
NW = 32
BPW = B // NW
GRP = 4
NGRP = BPW // GRP
NSEM = 12


def _sc_kernel(user_ids_h, pos_ids_h, neg_ids_h, user_t_h, movie_t_h,
               out_pos_h, out_neg_h,
               uidx_v, pidx_v, nidx_v,
               w_b0, w_b1, op_v, on_v, *sems_flat):
    wid = lax.axis_index("s") * 2 + lax.axis_index("c")

    pltpu.sync_copy(user_ids_h.at[pl.ds(wid * BPW, BPW)],
                    uidx_v.at[pl.ds(0, BPW)])
    pltpu.sync_copy(pos_ids_h.at[pl.ds(wid * BPW, BPW)],
                    pidx_v.at[pl.ds(0, BPW)])
    pltpu.sync_copy(neg_ids_h.at[pl.ds(wid * BPW, BPW)],
                    nidx_v.at[pl.ds(0, BPW)])

    bufs = (w_b0, w_b1)
    sems = (sems_flat[:NSEM], sems_flat[NSEM:])
    tabs = (user_t_h, movie_t_h, movie_t_h)
    idxs = (uidx_v, pidx_v, nidx_v)

    def fire_group(g, par):
        buf = bufs[par]
        psems = sems[par]
        vecs = [idxs[t][pl.ds(g * GRP, 16)] for t in range(3)]
        for t in range(3):
            for k in range(GRP):
                rid = vecs[t][k]
                col0 = pl.multiple_of((rid // 128) * 128, 128)
                j = t * GRP + k
                pltpu.async_copy(
                    tabs[t].at[pl.ds(0, EMB), pl.ds(col0, 128)],
                    buf.at[j],
                    psems[j % NSEM])

    def wait_group(par):
        buf = bufs[par]
        psems = sems[par]
        for j in range(3 * GRP):
            pltpu.make_async_copy(
                tabs[0].at[pl.ds(0, EMB), pl.ds(0, 128)],
                buf.at[j],
                psems[j % NSEM],
            ).wait()

    lo = lax.iota(jnp.int32, 16)
    hi = lo + 16

    def extract(buf, slot, cid):
        col = jnp.full((16,), cid, jnp.int32)
        slot_v = jnp.full((16,), slot, jnp.int32)
        a = plsc.load_gather(buf, [slot_v, lo, col])
        b = plsc.load_gather(buf, [slot_v, hi, col])
        return a, b

    def compute_group(g, par, accp, accn):
        buf = bufs[par]
        vecs = [idxs[t][pl.ds(g * GRP, 16)] for t in range(3)]
        for k in range(GRP):
            ua, ub = extract(buf, 0 * GRP + k, lax.rem(vecs[0][k], 128))
            pa, pb = extract(buf, 1 * GRP + k, lax.rem(vecs[1][k], 128))
            na, nb = extract(buf, 2 * GRP + k, lax.rem(vecs[2][k], 128))
            lane = lax.rem(g, 4) * GRP + k
            accp = jnp.where(lo == lane, jnp.sum(ua * pa + ub * pb), accp)
            accn = jnp.where(lo == lane, jnp.sum(ua * na + ub * nb), accn)
        is_last = lax.rem(g, 4) == 3

        @pl.when(is_last)
        def _():
            base = (g // 4) * 16
            op_v[pl.ds(base, 16)] = accp
            on_v[pl.ds(base, 16)] = accn

        keep = jnp.where(is_last, 0.0, 1.0)
        return accp * keep, accn * keep

    fire_group(0, 0)
    zeros = jnp.zeros((16,), jnp.float32)

    def pair(i, carry):
        accp, accn = carry
        g0 = i * 2
        fire_group(g0 + 1, 1)
        wait_group(0)
        accp, accn = compute_group(g0, 0, accp, accn)

        @pl.when(i + 1 < NGRP // 2)
        def _():
            fire_group(g0 + 2, 0)

        wait_group(1)
        accp, accn = compute_group(g0 + 1, 1, accp, accn)
        return accp, accn

    lax.fori_loop(0, NGRP // 2, pair, (zeros, zeros))

    out = pl.ds(wid * BPW, BPW)
    pltpu.sync_copy(op_v, out_pos_h.at[out])
    pltpu.sync_copy(on_v, out_neg_h.at[out])


@jax.jit
def kernel(user_ids, pos_ids, neg_ids, user_emb, movie_emb):
    uids = user_ids.astype(jnp.int32)
    pids = pos_ids.astype(jnp.int32)
    nids = neg_ids.astype(jnp.int32)

    mesh = plsc.VectorSubcoreMesh(
        core_axis_name="c", subcore_axis_name="s", num_cores=2, num_subcores=16
    )
    buf = pltpu.VMEM((3 * GRP, EMB, 128), jnp.float32)
    run = pl.kernel(
        _sc_kernel,
        out_type=(
            jax.ShapeDtypeStruct((B,), jnp.float32),
            jax.ShapeDtypeStruct((B,), jnp.float32),
        ),
        mesh=mesh,
        scratch_types=[
            pltpu.VMEM((BPW + 16,), jnp.int32),
            pltpu.VMEM((BPW + 16,), jnp.int32),
            pltpu.VMEM((BPW + 16,), jnp.int32),
            buf, buf,
            pltpu.VMEM((BPW,), jnp.float32),
            pltpu.VMEM((BPW,), jnp.float32),
        ] + [pltpu.SemaphoreType.DMA] * (2 * NSEM),
        compiler_params=pltpu.CompilerParams(needs_layout_passes=False),
    )
    return run(uids, pids, nids, user_emb.T, movie_emb.T)
